# baseline (speedup 1.0000x reference)
import functools

import jax
import jax.numpy as jnp
from jax import lax
from jax.experimental import pallas as pl
from jax.experimental.pallas import tpu as pltpu
from jax.experimental.pallas import tpu_sc as plsc

_NC = 1000
_NW = 32


def kernel(x):
    n0, n1 = x.shape
    rows_per_w = n0 // _NW
    zrow = jnp.zeros((n1, _NC), jnp.float32)
    mesh = plsc.VectorSubcoreMesh(core_axis_name="c", subcore_axis_name="s")

    @functools.partial(
        pl.kernel,
        out_type=jax.ShapeDtypeStruct((n0, n1, _NC), jnp.float32),
        mesh=mesh,
        scratch_types=[
            pltpu.VMEM((rows_per_w, n1), jnp.int32),
            pltpu.VMEM((n1, _NC), jnp.float32),
        ],
        compiler_params=pltpu.CompilerParams(
            use_tc_tiling_on_sc=False, needs_layout_passes=False),
    )
    def _sc(x_hbm, z_hbm, o_hbm, xv, buf):
        wid = lax.axis_index("s") * 2 + lax.axis_index("c")
        r0 = wid * rows_per_w
        pltpu.sync_copy(x_hbm.at[pl.ds(r0, rows_per_w), :], xv)
        pltpu.sync_copy(z_hbm, buf)
        s0 = lax.iota(jnp.int32, 16)
        s1 = s0 + (n1 - 16)
        ones = jnp.full((16,), 1.0, jnp.float32)
        zeros = jnp.zeros((16,), jnp.float32)

        def slice_body(j, carry):
            c0 = xv[j, 0:16]
            c1 = xv[j, pl.ds(n1 - 16, 16)]
            plsc.store_scatter(buf, [s0, c0], ones)
            plsc.store_scatter(buf, [s1, c1], ones)
            pltpu.sync_copy(buf, o_hbm.at[r0 + j])
            plsc.store_scatter(buf, [s0, c0], zeros)
            plsc.store_scatter(buf, [s1, c1], zeros)
            return carry

        lax.fori_loop(0, rows_per_w, slice_body, 0)

    return _sc(x, zrow)
